# baseline (speedup 1.0000x reference)
import functools

import jax
import jax.numpy as jnp
from jax import lax
from jax.experimental import pallas as pl
from jax.experimental.pallas import tpu as pltpu
from jax.experimental.pallas import tpu_sc as plsc

N = 10000
E = 320000
D = 128
NC = 2
NS = 16
NW = NC * NS
K = 128
CPT = 80
E_PAD = NW * CPT * K
ROWS2D = E_PAD // K
DUMMY = N
NT = 10240
RPT = NT // NS
BM = 1024

_mesh = plsc.VectorSubcoreMesh(core_axis_name="c", subcore_axis_name="s")


@functools.partial(
    pl.kernel,
    out_type=jax.ShapeDtypeStruct((NC, NT, 8), jnp.float32),
    mesh=_mesh,
    scratch_types=[
        pltpu.VMEM((CPT, K), jnp.int32),
        pltpu.VMEM((K, 8), jnp.float32),
        pltpu.VMEM_SHARED((NT, 8), jnp.float32),
        pltpu.SemaphoreType.DMA,
    ],
)
def _deg_kernel(dst_hbm, ones_hbm, out_hbm, dst_v, ones_v, acc_sh, sem):
    c = lax.axis_index("c")
    s = lax.axis_index("s")
    wid = c * NS + s
    pltpu.sync_copy(dst_hbm.at[pl.ds(wid * CPT, CPT)], dst_v)
    pltpu.sync_copy(ones_hbm.at[pl.ds(0, K)], ones_v)
    pltpu.sync_copy(ones_hbm.at[pl.ds(s * RPT, RPT)],
                    acc_sh.at[pl.ds(s * RPT, RPT)])
    plsc.subcore_barrier()

    def body(j, carry):
        pltpu.sync_copy(ones_v, acc_sh.at[dst_v.at[j]], add=True)
        return carry

    lax.fori_loop(0, CPT, body, 0)
    plsc.subcore_barrier()
    pltpu.sync_copy(acc_sh.at[pl.ds(s * RPT, RPT)],
                    out_hbm.at[c, pl.ds(s * RPT, RPT)])


GRP = 16
CPT2 = ROWS2D // NS
NGRP2 = CPT2 // GRP
HALF = NT // 2
RPH = HALF // NS
HPAD = HALF + K


@functools.partial(
    pl.kernel,
    out_type=jax.ShapeDtypeStruct((NT, D), jnp.float32),
    mesh=_mesh,
    scratch_types=[
        pltpu.VMEM((GRP, K), jnp.int32),
        pltpu.VMEM((GRP, K), jnp.int32),
        pltpu.VMEM((K, D), jnp.float32),
        pltpu.VMEM((K, D), jnp.float32),
        pltpu.VMEM((K,), jnp.int32),
        pltpu.VMEM((K,), jnp.int32),
        pltpu.VMEM_SHARED((HPAD, D), jnp.float32),
        pltpu.SemaphoreType.DMA,
        pltpu.SemaphoreType.DMA,
    ],
)
def _agg_kernel(u_hbm, src_hbm, dst_hbm, out_hbm,
                src_v, dst_v, buf0, buf1, dstl0, dstl1, acc_sh, sem0, sem1):
    c = lax.axis_index("c")
    s = lax.axis_index("s")
    base = c * HALF
    pltpu.sync_copy(u_hbm.at[pl.ds(base + s * RPH, RPH)],
                    acc_sh.at[pl.ds(s * RPH, RPH)])
    plsc.subcore_barrier()

    def adjust(j, dstl):
        for t in range(K // 16):
            v = dst_v[j, pl.ds(t * 16, 16)] - base
            ok = (v >= 0) & (v < HALF)
            dstl[pl.ds(t * 16, 16)] = jnp.where(ok, v, HALF)

    def group(grp, carry):
        g0 = s * CPT2 + grp * GRP
        pltpu.sync_copy(src_hbm.at[pl.ds(g0, GRP)], src_v)
        pltpu.sync_copy(dst_hbm.at[pl.ds(g0, GRP)], dst_v)
        pltpu.async_copy(u_hbm.at[src_v.at[0]], buf0, sem0)

        def body(p, carry2):
            j0 = 2 * p
            j1 = j0 + 1
            pltpu.async_copy(u_hbm.at[src_v.at[j1]], buf1, sem1)
            adjust(j0, dstl0)
            pltpu.make_async_copy(u_hbm.at[src_v.at[j0]], buf0, sem0).wait()
            pltpu.sync_copy(buf0, acc_sh.at[dstl0], add=True)

            @pl.when(p < GRP // 2 - 1)
            def _():
                pltpu.async_copy(u_hbm.at[src_v.at[j0 + 2]], buf0, sem0)

            adjust(j1, dstl1)
            pltpu.make_async_copy(u_hbm.at[src_v.at[j1]], buf1, sem1).wait()
            pltpu.sync_copy(buf1, acc_sh.at[dstl1], add=True)
            return carry2

        lax.fori_loop(0, GRP // 2, body, 0)
        return carry

    lax.fori_loop(0, NGRP2, group, 0)
    plsc.subcore_barrier()
    pltpu.sync_copy(acc_sh.at[pl.ds(s * RPH, RPH)],
                    out_hbm.at[pl.ds(base + s * RPH, RPH)])


def _g_body(d0_ref, d1_ref, g_ref):
    g_ref[...] = lax.rsqrt(d0_ref[...] + d1_ref[...] - 1.0)


def _u_body(h_ref, w_ref, g_ref, u_ref):
    p = jnp.dot(h_ref[...], w_ref[...], preferred_element_type=jnp.float32)
    u_ref[...] = p * g_ref[:, 0:1]


def _post_body(s_ref, g_ref, b_ref, o_ref):
    agg = g_ref[:, 0:1] * s_ref[...] + b_ref[...]
    o_ref[...] = jnp.maximum(agg, 0.0)


def _row_spec(cols):
    return pl.BlockSpec((BM, cols), lambda i: (i, 0))


def _full_spec(rows, cols):
    return pl.BlockSpec((rows, cols), lambda i: (0, 0))


def kernel(x, edge_index, W1, b1, W2, b2):
    src = edge_index[0].astype(jnp.int32)
    dst = edge_index[1].astype(jnp.int32)
    pad = E_PAD - E
    src2d = jnp.concatenate(
        [src, jnp.zeros((pad,), jnp.int32)]).reshape(ROWS2D, K)
    dst2d = jnp.concatenate(
        [dst, jnp.full((pad,), DUMMY, jnp.int32)]).reshape(ROWS2D, K)
    ones_rows = jnp.ones((NT, 8), jnp.float32)
    xp = jnp.concatenate([x, jnp.zeros((NT - N, D), jnp.float32)])
    Ws = jnp.stack([W1, W2])
    bs = jnp.stack([b1, b2]).reshape(2, 1, D)

    degp = _deg_kernel(dst2d, ones_rows)
    g8 = pl.pallas_call(
        _g_body,
        grid=(1,),
        in_specs=[_full_spec(NT, 8), _full_spec(NT, 8)],
        out_specs=_full_spec(NT, 8),
        out_shape=jax.ShapeDtypeStruct((NT, 8), jnp.float32),
    )(degp[0], degp[1])

    grid = (NT // BM,)
    u_call = pl.pallas_call(
        _u_body,
        grid=grid,
        in_specs=[_row_spec(D), _full_spec(D, D), _row_spec(8)],
        out_specs=_row_spec(D),
        out_shape=jax.ShapeDtypeStruct((NT, D), jnp.float32),
    )
    post_call = pl.pallas_call(
        _post_body,
        grid=grid,
        in_specs=[_row_spec(D), _row_spec(8), _full_spec(1, D)],
        out_specs=_row_spec(D),
        out_shape=jax.ShapeDtypeStruct((NT, D), jnp.float32),
    )

    def layer(h, wb):
        W, b = wb
        u = u_call(h, W, g8)
        s = _agg_kernel(u, src2d, dst2d)
        return post_call(s, g8, b), None

    h, _ = lax.scan(layer, xp, (Ws, bs))
    return h[:N]


# speedup vs baseline: 4.4566x; 4.4566x over previous
import functools

import jax
import jax.numpy as jnp
from jax import lax
from jax.experimental import pallas as pl
from jax.experimental.pallas import tpu as pltpu
from jax.experimental.pallas import tpu_sc as plsc

N = 10000
E = 320000
D = 128
NC = 2
NS = 16
NW = NC * NS
K = 128
CPT = 80
E_PAD = NW * CPT * K
ROWS2D = E_PAD // K
DUMMY = N
NT = 10240
RPT = NT // NS
BM = 1024

_mesh = plsc.VectorSubcoreMesh(core_axis_name="c", subcore_axis_name="s")


@functools.partial(
    pl.kernel,
    out_type=jax.ShapeDtypeStruct((NC, NT, 8), jnp.float32),
    mesh=_mesh,
    scratch_types=[
        pltpu.VMEM((CPT, K), jnp.int32),
        pltpu.VMEM((K, 8), jnp.float32),
        pltpu.VMEM_SHARED((NT, 8), jnp.float32),
        pltpu.SemaphoreType.DMA,
    ],
)
def _deg_kernel(dst_hbm, ones_hbm, out_hbm, dst_v, ones_v, acc_sh, sem):
    c = lax.axis_index("c")
    s = lax.axis_index("s")
    wid = c * NS + s
    pltpu.sync_copy(dst_hbm.at[pl.ds(wid * CPT, CPT)], dst_v)
    pltpu.sync_copy(ones_hbm.at[pl.ds(0, K)], ones_v)
    pltpu.sync_copy(ones_hbm.at[pl.ds(s * RPT, RPT)],
                    acc_sh.at[pl.ds(s * RPT, RPT)])
    plsc.subcore_barrier()

    def body(j, carry):
        pltpu.sync_copy(ones_v, acc_sh.at[dst_v.at[j]], add=True)
        return carry

    lax.fori_loop(0, CPT, body, 0)
    fidx = lax.iota(jnp.int32, 16) + (NT - 16)
    pltpu.sync_copy(ones_v.at[pl.ds(0, 16)], acc_sh.at[fidx], add=True)
    plsc.subcore_barrier()
    pl.delay(2000)
    pltpu.sync_copy(acc_sh.at[pl.ds(s * RPT, RPT)],
                    out_hbm.at[c, pl.ds(s * RPT, RPT)])


GRP = 16
NGRP = CPT // GRP


@functools.partial(
    pl.kernel,
    out_type=jax.ShapeDtypeStruct((NC, NT, D), jnp.float32),
    mesh=_mesh,
    scratch_types=[
        pltpu.VMEM((GRP, K), jnp.int32),
        pltpu.VMEM((GRP, K), jnp.int32),
        pltpu.VMEM((K, D), jnp.float32),
        pltpu.VMEM((K, D), jnp.float32),
        pltpu.VMEM_SHARED((NT, D), jnp.float32),
        pltpu.SemaphoreType.DMA,
        pltpu.SemaphoreType.DMA,
    ],
)
def _agg_kernel(u_hbm, src_hbm, dst_hbm, out_hbm,
                src_v, dst_v, buf0, buf1, acc_sh, sem0, sem1):
    c = lax.axis_index("c")
    s = lax.axis_index("s")
    wid = c * NS + s
    pltpu.sync_copy(u_hbm.at[pl.ds(s * RPT, RPT)],
                    acc_sh.at[pl.ds(s * RPT, RPT)])
    plsc.subcore_barrier()

    def group(grp, carry):
        g0 = wid * CPT + grp * GRP
        pltpu.sync_copy(src_hbm.at[pl.ds(g0, GRP)], src_v)
        pltpu.sync_copy(dst_hbm.at[pl.ds(g0, GRP)], dst_v)
        pltpu.async_copy(u_hbm.at[src_v.at[0]], buf0, sem0)

        def body(p, carry2):
            j0 = 2 * p
            j1 = j0 + 1
            pltpu.async_copy(u_hbm.at[src_v.at[j1]], buf1, sem1)
            pltpu.make_async_copy(u_hbm.at[src_v.at[j0]], buf0, sem0).wait()
            pltpu.sync_copy(buf0, acc_sh.at[dst_v.at[j0]], add=True)

            @pl.when(p < GRP // 2 - 1)
            def _():
                pltpu.async_copy(u_hbm.at[src_v.at[j0 + 2]], buf0, sem0)

            pltpu.make_async_copy(u_hbm.at[src_v.at[j1]], buf1, sem1).wait()
            pltpu.sync_copy(buf1, acc_sh.at[dst_v.at[j1]], add=True)
            return carry2

        lax.fori_loop(0, GRP // 2, body, 0)
        return carry

    lax.fori_loop(0, NGRP, group, 0)
    fidx = lax.iota(jnp.int32, 16) + (NT - 16)
    pltpu.sync_copy(buf0.at[pl.ds(0, 16)], acc_sh.at[fidx], add=True)
    plsc.subcore_barrier()
    pl.delay(2000)
    pltpu.sync_copy(acc_sh.at[pl.ds(s * RPT, RPT)],
                    out_hbm.at[c, pl.ds(s * RPT, RPT)])


def _g_body(d0_ref, d1_ref, g_ref):
    g_ref[...] = lax.rsqrt(d0_ref[...] + d1_ref[...] - 1.0)


def _u_body(h_ref, w_ref, g_ref, u_ref):
    p = jnp.dot(h_ref[...], w_ref[...], preferred_element_type=jnp.float32)
    u_ref[...] = p * g_ref[:, 0:1]


def _post_body(s0_ref, s1_ref, u_ref, g_ref, b_ref, o_ref):
    g = g_ref[:, 0:1]
    agg = g * (s0_ref[...] + s1_ref[...] - u_ref[...]) + b_ref[...]
    o_ref[...] = jnp.maximum(agg, 0.0)


def _row_spec(cols):
    return pl.BlockSpec((BM, cols), lambda i: (i, 0))


def _full_spec(rows, cols):
    return pl.BlockSpec((rows, cols), lambda i: (0, 0))


def kernel(x, edge_index, W1, b1, W2, b2):
    src = edge_index[0].astype(jnp.int32)
    dst = edge_index[1].astype(jnp.int32)
    pad = E_PAD - E
    spread = jnp.arange(pad, dtype=jnp.int32) % K
    src2d = jnp.concatenate([src, spread]).reshape(ROWS2D, K)
    dst2d = jnp.concatenate([dst, DUMMY + spread]).reshape(ROWS2D, K)
    ones_rows = jnp.ones((NT, 8), jnp.float32)
    xp = jnp.concatenate([x, jnp.zeros((NT - N, D), jnp.float32)])
    Ws = jnp.stack([W1, W2])
    bs = jnp.stack([b1, b2]).reshape(2, 1, D)

    degp = _deg_kernel(dst2d, ones_rows)
    g8 = pl.pallas_call(
        _g_body,
        grid=(1,),
        in_specs=[_full_spec(NT, 8), _full_spec(NT, 8)],
        out_specs=_full_spec(NT, 8),
        out_shape=jax.ShapeDtypeStruct((NT, 8), jnp.float32),
    )(degp[0], degp[1])

    grid = (NT // BM,)
    u_call = pl.pallas_call(
        _u_body,
        grid=grid,
        in_specs=[_row_spec(D), _full_spec(D, D), _row_spec(8)],
        out_specs=_row_spec(D),
        out_shape=jax.ShapeDtypeStruct((NT, D), jnp.float32),
    )
    post_call = pl.pallas_call(
        _post_body,
        grid=grid,
        in_specs=[_row_spec(D), _row_spec(D), _row_spec(D), _row_spec(8),
                  _full_spec(1, D)],
        out_specs=_row_spec(D),
        out_shape=jax.ShapeDtypeStruct((NT, D), jnp.float32),
    )

    def layer(h, wb):
        W, b = wb
        u = u_call(h, W, g8)
        s = _agg_kernel(u, src2d, dst2d)
        return post_call(s[0], s[1], u, g8, b), None

    h, _ = lax.scan(layer, xp, (Ws, bs))
    return h[:N]
